# baseline (speedup 1.0000x reference)
import functools

import jax
import jax.numpy as jnp
from jax import lax
from jax.experimental import pallas as pl
from jax.experimental.pallas import tpu as pltpu
from jax.experimental.pallas import tpu_sc as plsc

N_NODES = 50000
N_EDGES = 1600000
NPAD = 50176
NW = 32
NGRP = N_EDGES // 128
CHUNK = 2048
NCH = 24
TAIL = 896


def _sc_pass(kinds):
    mesh = plsc.VectorSubcoreMesh(core_axis_name="c", subcore_axis_name="s")
    out_type = [jax.ShapeDtypeStruct((NW, NPAD), jnp.float32)
                for _ in kinds]
    if "p2" in kinds:
        out_type.append(jax.ShapeDtypeStruct((NW, 16), jnp.float32))
    out_type = tuple(out_type)

    scratch = [
        pltpu.VMEM((NPAD,), jnp.float32),
        pltpu.VMEM_SHARED((NPAD,), jnp.float32),
        pltpu.VMEM((NPAD,), jnp.float32),
        pltpu.VMEM((2, CHUNK), jnp.int32),
        pltpu.VMEM((2, CHUNK), jnp.int32),
        pltpu.VMEM((2, TAIL), jnp.int32),
        pltpu.VMEM((16,), jnp.float32),
        pltpu.SemaphoreType.DMA,
        pltpu.SemaphoreType.DMA,
        pltpu.SemaphoreType.DMA,
    ]

    @functools.partial(pl.kernel, out_type=out_type, mesh=mesh,
                       scratch_types=scratch,
                       compiler_params=pltpu.CompilerParams(
                           needs_layout_passes=False))
    def body(node_hbm, ei_hbm, *rest):
        outs = {k: rest[i] for i, k in enumerate(kinds)}
        rest = rest[len(kinds):]
        if "p2" in kinds:
            oute_hbm = rest[0]
            rest = rest[1:]
        (node_v, sh_node, acc_v, b0, b1, tb, e_v, sem0, sem1, semt) = rest
        bufs = (b0, b1)
        sems = (sem0, sem1)

        sid = lax.axis_index("s")
        wid = sid * 2 + lax.axis_index("c")
        start_g = (wid * NGRP) // NW
        end_g = ((wid + 1) * NGRP) // NW
        ebase = start_g * 128
        n_tail = (end_g - start_g) * 128 - NCH * CHUNK

        def start(c):
            slot = c % 2
            return pltpu.async_copy(
                ei_hbm.at[:, pl.ds(ebase + c * CHUNK, CHUNK)],
                bufs[slot], sems[slot])

        def start_tail():
            return pltpu.async_copy(
                ei_hbm.at[:, pl.ds(ebase + NCH * CHUNK, TAIL)],
                tb, semt)

        handles = [start(0), None]

        @pl.when(sid == 0)
        def _():
            pltpu.sync_copy(node_hbm, sh_node)
        zeros16 = jnp.zeros((16,), jnp.float32)
        plsc.subcore_barrier()
        pltpu.sync_copy(sh_node, node_v)

        def process(kind, buf, ub, unroll):
            if kind == "p2":
                def edge_body(j, ecarry):
                    s = buf[0, pl.ds(j, 16)]
                    d = buf[1, pl.ds(j, 16)]
                    vs = plsc.load_gather(node_v, [s])
                    vd = plsc.load_gather(node_v, [d])
                    plsc.addupdate_scatter(acc_v, [s], vd)
                    plsc.addupdate_scatter(acc_v, [d], vs)
                    return ecarry + vs * vd
                esum = plsc.parallel_loop(
                    0, ub, step=16, unroll=unroll,
                    carry=jnp.zeros((16,), jnp.float32))(edge_body)
                e_v[...] = e_v[...] + esum
            else:
                @plsc.parallel_loop(0, ub, step=16, unroll=unroll)
                def _(j):
                    s = buf[0, pl.ds(j, 16)]
                    d = buf[1, pl.ds(j, 16)]
                    if kind == "p1":
                        us = plsc.load_gather(node_v, [s])
                        ud = plsc.load_gather(node_v, [d])
                        plsc.addupdate_scatter(acc_v, [d], us * ud)
                    elif kind == "pb":
                        us = plsc.load_gather(node_v, [s])
                        plsc.addupdate_scatter(acc_v, [d], us)
                    else:
                        wd = plsc.load_gather(node_v, [d])
                        plsc.addupdate_scatter(acc_v, [s], wd)

        for kind in kinds:
            @plsc.parallel_loop(0, NPAD, step=16, unroll=8)
            def _(i):
                acc_v[pl.ds(i, 16)] = zeros16
            e_v[...] = zeros16

            for c in range(NCH):
                handles[(c + 1) % 2] = (start(c + 1) if c + 1 < NCH
                                        else start_tail())
                handles[c % 2].wait()
                process(kind, bufs[c % 2], CHUNK, 8)
            handles[NCH % 2].wait()
            process(kind, tb, n_tail, 2)
            if kind != kinds[-1]:
                handles = [start(0), None]

            pltpu.sync_copy(acc_v, outs[kind].at[wid])
            if kind == "p2":
                pltpu.sync_copy(e_v, oute_hbm.at[wid])

    return body


_sc_p1pb = _sc_pass(("p1", "pb"))
_sc_p2 = _sc_pass(("p2",))
_sc_p3 = _sc_pass(("p3",))


def _full_spec(shape):
    return pl.BlockSpec(shape, lambda: (0,) * len(shape))


def _tc_u(x2):
    def body(x_ref, u_ref):
        xp = x_ref[...] + 1.0
        u_ref[...] = jnp.concatenate(
            [xp * jnp.sqrt(xp),
             jnp.zeros((1, NPAD - N_NODES), jnp.float32)], axis=1)
    return pl.pallas_call(
        body,
        out_shape=jax.ShapeDtypeStruct((1, NPAD), jnp.float32),
        in_specs=[_full_spec((1, N_NODES))],
        out_specs=_full_spec((1, NPAD)),
    )(x2)


def _tc_stage1(py1, pB):
    def body(py1_ref, pB_ref, v_ref, fp1_ref, B_ref):
        y1 = jnp.sum(py1_ref[...], axis=0, keepdims=True)
        yp = y1 + 1.0
        r = jnp.sqrt(yp)
        v_ref[...] = yp * r
        fp1_ref[...] = 1.5 * r
        B_ref[...] = jnp.sum(pB_ref[...], axis=0, keepdims=True)
    return pl.pallas_call(
        body,
        out_shape=[jax.ShapeDtypeStruct((1, NPAD), jnp.float32)] * 3,
        in_specs=[_full_spec((NW, NPAD))] * 2,
        out_specs=[_full_spec((1, NPAD))] * 3,
    )(py1, pB)


def _tc_stage2(ph1, pe, fp1, u):
    def body(ph1_ref, pe_ref, fp1_ref, u_ref, g1_ref, w_ref, en_ref):
        h1 = jnp.sum(ph1_ref[...], axis=0, keepdims=True)
        g1 = fp1_ref[...] * h1
        g1_ref[...] = g1
        w_ref[...] = g1 * u_ref[...]
        en_ref[...] = jnp.full((1, 128), jnp.sum(pe_ref[...]), jnp.float32)
    return pl.pallas_call(
        body,
        out_shape=[jax.ShapeDtypeStruct((1, NPAD), jnp.float32),
                   jax.ShapeDtypeStruct((1, NPAD), jnp.float32),
                   jax.ShapeDtypeStruct((1, 128), jnp.float32)],
        in_specs=[_full_spec((NW, NPAD)), _full_spec((NW, 16)),
                  _full_spec((1, NPAD)), _full_spec((1, NPAD))],
        out_specs=[_full_spec((1, NPAD))] * 2 + [_full_spec((1, 128))],
    )(ph1, pe, fp1, u)


def _tc_stage3(pA, g1, B, x2):
    def body(pA_ref, g1_ref, B_ref, x_ref, f_ref):
        A = jnp.sum(pA_ref[...], axis=0, keepdims=True)[:, :N_NODES]
        f_ref[...] = (1.5 * jnp.sqrt(x_ref[...] + 1.0)
                      * (A + g1_ref[:, :N_NODES] * B_ref[:, :N_NODES]))
    return pl.pallas_call(
        body,
        out_shape=jax.ShapeDtypeStruct((1, N_NODES), jnp.float32),
        in_specs=[_full_spec((NW, NPAD))] + [_full_spec((1, NPAD))] * 2
        + [_full_spec((1, N_NODES))],
        out_specs=_full_spec((1, N_NODES)),
    )(pA, g1, B, x2)


def kernel(atomic_numbers, edge_index):
    x = atomic_numbers.astype(jnp.float32)
    ei = edge_index.astype(jnp.int32)

    x2 = x.reshape(1, N_NODES)

    u2 = _tc_u(x2)
    u1 = u2.reshape(NPAD)

    py1, pB = _sc_p1pb(u1, ei)
    v2, fp1, B2 = _tc_stage1(py1, pB)

    ph1, pe = _sc_p2(v2.reshape(NPAD), ei)
    g1_2, w2, en = _tc_stage2(ph1, pe, fp1, u2)

    (pA,) = _sc_p3(w2.reshape(NPAD), ei)
    forces2 = _tc_stage3(pA, g1_2, B2, x2)

    energy = en[0, 0].reshape(1)
    forces = forces2.reshape(N_NODES)
    return (energy, forces)


# speedup vs baseline: 1.0129x; 1.0129x over previous
import functools

import jax
import jax.numpy as jnp
from jax import lax
from jax.experimental import pallas as pl
from jax.experimental.pallas import tpu as pltpu
from jax.experimental.pallas import tpu_sc as plsc

N_NODES = 50000
N_EDGES = 1600000
NPAD = 50176
NW = 32
NGRP = N_EDGES // 128
CHUNK = 2048
NCH = 24
TAIL = 896


def _sc_pass(kinds):
    mesh = plsc.VectorSubcoreMesh(core_axis_name="c", subcore_axis_name="s")
    out_type = [jax.ShapeDtypeStruct((NW, NPAD), jnp.float32)
                for _ in kinds]
    if "p2" in kinds:
        out_type.append(jax.ShapeDtypeStruct((NW, 16), jnp.float32))
    out_type = tuple(out_type)

    scratch = [
        pltpu.VMEM((NPAD,), jnp.float32),
        pltpu.VMEM_SHARED((NPAD,), jnp.float32),
        pltpu.VMEM((NPAD,), jnp.float32),
        pltpu.VMEM((2, CHUNK), jnp.int32),
        pltpu.VMEM((2, CHUNK), jnp.int32),
        pltpu.VMEM((2, TAIL), jnp.int32),
        pltpu.VMEM((16,), jnp.float32),
        pltpu.SemaphoreType.DMA,
        pltpu.SemaphoreType.DMA,
        pltpu.SemaphoreType.DMA,
    ]

    @functools.partial(pl.kernel, out_type=out_type, mesh=mesh,
                       scratch_types=scratch,
                       compiler_params=pltpu.CompilerParams(
                           needs_layout_passes=False))
    def body(node_hbm, ei_hbm, *rest):
        outs = {k: rest[i] for i, k in enumerate(kinds)}
        rest = rest[len(kinds):]
        if "p2" in kinds:
            oute_hbm = rest[0]
            rest = rest[1:]
        (node_v, sh_node, acc_v, b0, b1, tb, e_v, sem0, sem1, semt) = rest
        bufs = (b0, b1)
        sems = (sem0, sem1)

        sid = lax.axis_index("s")
        wid = sid * 2 + lax.axis_index("c")
        start_g = (wid * NGRP) // NW
        end_g = ((wid + 1) * NGRP) // NW
        ebase = start_g * 128
        n_tail = (end_g - start_g) * 128 - NCH * CHUNK

        def start(c):
            slot = c % 2
            return pltpu.async_copy(
                ei_hbm.at[:, pl.ds(ebase + c * CHUNK, CHUNK)],
                bufs[slot], sems[slot])

        def start_tail():
            return pltpu.async_copy(
                ei_hbm.at[:, pl.ds(ebase + NCH * CHUNK, TAIL)],
                tb, semt)

        handles = [start(0), None]

        @pl.when(sid == 0)
        def _():
            pltpu.sync_copy(node_hbm, sh_node)
        zeros16 = jnp.zeros((16,), jnp.float32)
        plsc.subcore_barrier()
        pltpu.sync_copy(sh_node, node_v)

        def process(kind, buf, ub, unroll):
            if kind == "p2":
                def edge_body(j, ecarry):
                    s = buf[0, pl.ds(j, 16)]
                    d = buf[1, pl.ds(j, 16)]
                    vs = plsc.load_gather(node_v, [s])
                    vd = plsc.load_gather(node_v, [d])
                    plsc.addupdate_scatter(acc_v, [s], vd)
                    plsc.addupdate_scatter(acc_v, [d], vs)
                    return ecarry + vs * vd
                esum = plsc.parallel_loop(
                    0, ub, step=16, unroll=unroll,
                    carry=jnp.zeros((16,), jnp.float32))(edge_body)
                e_v[...] = e_v[...] + esum
            else:
                @plsc.parallel_loop(0, ub, step=16, unroll=unroll)
                def _(j):
                    s = buf[0, pl.ds(j, 16)]
                    d = buf[1, pl.ds(j, 16)]
                    if kind == "p1":
                        us = plsc.load_gather(node_v, [s])
                        ud = plsc.load_gather(node_v, [d])
                        plsc.addupdate_scatter(acc_v, [d], us * ud)
                    elif kind == "pb":
                        us = plsc.load_gather(node_v, [s])
                        plsc.addupdate_scatter(acc_v, [d], us)
                    else:
                        wd = plsc.load_gather(node_v, [d])
                        plsc.addupdate_scatter(acc_v, [s], wd)

        for kind in kinds:
            @plsc.parallel_loop(0, NPAD, step=16, unroll=8)
            def _(i):
                acc_v[pl.ds(i, 16)] = zeros16
            e_v[...] = zeros16

            for c in range(NCH):
                handles[(c + 1) % 2] = (start(c + 1) if c + 1 < NCH
                                        else start_tail())
                handles[c % 2].wait()
                process(kind, bufs[c % 2], CHUNK, 8)
            handles[NCH % 2].wait()
            process(kind, tb, n_tail, 2)
            if kind != kinds[-1]:
                handles = [start(0), None]

            pltpu.sync_copy(acc_v, outs[kind].at[wid])
            if kind == "p2":
                pltpu.sync_copy(e_v, oute_hbm.at[wid])

    return body


_sc_p1pb = _sc_pass(("p1", "pb"))
_sc_p2 = _sc_pass(("p2",))
_sc_p3 = _sc_pass(("p3",))


def _full_spec(shape):
    return pl.BlockSpec(shape, lambda: (0,) * len(shape))


def _tc_u(x2):
    def body(x_ref, u_ref):
        xp = x_ref[...] + 1.0
        u_ref[...] = xp * jnp.sqrt(xp)
    return pl.pallas_call(
        body,
        out_shape=jax.ShapeDtypeStruct((1, NPAD), jnp.float32),
        in_specs=[_full_spec((1, NPAD))],
        out_specs=_full_spec((1, NPAD)),
    )(x2)


def _tc_stage1(py1, pB):
    def body(py1_ref, pB_ref, v_ref, fp1_ref, B_ref):
        y1 = jnp.sum(py1_ref[...], axis=0, keepdims=True)
        yp = y1 + 1.0
        r = jnp.sqrt(yp)
        v_ref[...] = yp * r
        fp1_ref[...] = 1.5 * r
        B_ref[...] = jnp.sum(pB_ref[...], axis=0, keepdims=True)
    return pl.pallas_call(
        body,
        out_shape=[jax.ShapeDtypeStruct((1, NPAD), jnp.float32)] * 3,
        in_specs=[_full_spec((NW, NPAD))] * 2,
        out_specs=[_full_spec((1, NPAD))] * 3,
    )(py1, pB)


def _tc_stage2(ph1, pe, fp1, u):
    def body(ph1_ref, pe_ref, fp1_ref, u_ref, g1_ref, w_ref, en_ref):
        h1 = jnp.sum(ph1_ref[...], axis=0, keepdims=True)
        g1 = fp1_ref[...] * h1
        g1_ref[...] = g1
        w_ref[...] = g1 * u_ref[...]
        en_ref[...] = jnp.full((1, 128), jnp.sum(pe_ref[...]), jnp.float32)
    return pl.pallas_call(
        body,
        out_shape=[jax.ShapeDtypeStruct((1, NPAD), jnp.float32),
                   jax.ShapeDtypeStruct((1, NPAD), jnp.float32),
                   jax.ShapeDtypeStruct((1, 128), jnp.float32)],
        in_specs=[_full_spec((NW, NPAD)), _full_spec((NW, 16)),
                  _full_spec((1, NPAD)), _full_spec((1, NPAD))],
        out_specs=[_full_spec((1, NPAD))] * 2 + [_full_spec((1, 128))],
    )(ph1, pe, fp1, u)


def _tc_stage3(pA, g1, B, x2):
    def body(pA_ref, g1_ref, B_ref, x_ref, f_ref):
        A = jnp.sum(pA_ref[...], axis=0, keepdims=True)
        f_ref[...] = (1.5 * jnp.sqrt(x_ref[...] + 1.0)
                      * (A + g1_ref[...] * B_ref[...]))
    return pl.pallas_call(
        body,
        out_shape=jax.ShapeDtypeStruct((1, NPAD), jnp.float32),
        in_specs=[_full_spec((NW, NPAD))] + [_full_spec((1, NPAD))] * 3,
        out_specs=_full_spec((1, NPAD)),
    )(pA, g1, B, x2)


def kernel(atomic_numbers, edge_index):
    x = atomic_numbers.astype(jnp.float32)
    ei = edge_index.astype(jnp.int32)

    x_pad = jnp.pad(x, (0, NPAD - N_NODES))
    x2 = x_pad.reshape(1, NPAD)

    u2 = _tc_u(x2)
    u1 = u2.reshape(NPAD)

    py1, pB = _sc_p1pb(u1, ei)
    v2, fp1, B2 = _tc_stage1(py1, pB)

    ph1, pe = _sc_p2(v2.reshape(NPAD), ei)
    g1_2, w2, en = _tc_stage2(ph1, pe, fp1, u2)

    (pA,) = _sc_p3(w2.reshape(NPAD), ei)
    forces2 = _tc_stage3(pA, g1_2, B2, x2)

    energy = en[0, 0].reshape(1)
    forces = forces2.reshape(NPAD)[:N_NODES]
    return (energy, forces)
